# baseline (speedup 1.0000x reference)
import functools

import jax
import jax.numpy as jnp
from jax.experimental import pallas as pl
from jax.experimental.pallas import tpu as pltpu


def _se_body(x_ref, w_ref, b_ref, o_ref, *, inv_hw):
    x = x_ref[...]
    pooled = jnp.sum(x, axis=(2, 3)) * inv_hw
    logits = jax.lax.dot_general(
        pooled, w_ref[...], (((1,), (1,)), ((), ())),
        preferred_element_type=jnp.float32,
    ) + b_ref[...]
    gate = jax.nn.sigmoid(logits)
    o_ref[...] = x * gate[..., None, None]


def kernel(x, weight, bias):
    B, C, H, W = x.shape
    bblk = 1
    w = jnp.asarray(weight).reshape(C, C)
    b_row = jnp.asarray(bias).reshape(1, C)

    return pl.pallas_call(
        functools.partial(_se_body, inv_hw=1.0 / (H * W)),
        out_shape=jax.ShapeDtypeStruct((B, C, H, W), x.dtype),
        grid=(B // bblk,),
        in_specs=[
            pl.BlockSpec((bblk, C, H, W), lambda b: (b, 0, 0, 0)),
            pl.BlockSpec((C, C), lambda b: (0, 0)),
            pl.BlockSpec((1, C), lambda b: (0, 0)),
        ],
        out_specs=pl.BlockSpec((bblk, C, H, W), lambda b: (b, 0, 0, 0)),
        compiler_params=pltpu.CompilerParams(
            dimension_semantics=("parallel",),
            vmem_limit_bytes=56 << 20,
        ),
    )(x, w, b_row)


# speedup vs baseline: 1.2712x; 1.2712x over previous
import functools

import jax
import jax.numpy as jnp
from jax.experimental import pallas as pl
from jax.experimental.pallas import tpu as pltpu


def _se_body(x_ref, w_ref, b_ref, o_ref, *, c, bblk, inv_hw):
    x = x_ref[...]
    cols = [
        jnp.sum(x[i * c:(i + 1) * c], axis=(1, 2), keepdims=True)[:, :, 0]
        for i in range(bblk)
    ]
    pooled = jnp.concatenate(cols, axis=1) * inv_hw if bblk > 1 else cols[0] * inv_hw
    logits = jax.lax.dot_general(
        w_ref[...], pooled, (((1,), (0,)), ((), ())),
        preferred_element_type=jnp.float32,
    ) + b_ref[...]
    gate = jax.nn.sigmoid(logits)
    for i in range(bblk):
        o_ref[i * c:(i + 1) * c] = x[i * c:(i + 1) * c] * gate[:, i:i + 1, None]


def kernel(x, weight, bias):
    B, C, H, W = x.shape
    HW = H * W
    S = HW // 128
    bblk = 4 if B % 4 == 0 else 1
    xt = x.reshape(B * C, S, 128)
    w = jnp.asarray(weight).reshape(C, C)
    b_col = jnp.asarray(bias).reshape(C, 1)

    out = pl.pallas_call(
        functools.partial(_se_body, c=C, bblk=bblk, inv_hw=1.0 / HW),
        out_shape=jax.ShapeDtypeStruct((B * C, S, 128), x.dtype),
        grid=(B // bblk,),
        in_specs=[
            pl.BlockSpec((bblk * C, S, 128), lambda b: (b, 0, 0)),
            pl.BlockSpec((C, C), lambda b: (0, 0)),
            pl.BlockSpec((C, 1), lambda b: (0, 0)),
        ],
        out_specs=pl.BlockSpec((bblk * C, S, 128), lambda b: (b, 0, 0)),
        compiler_params=pltpu.CompilerParams(
            dimension_semantics=("parallel",),
            vmem_limit_bytes=56 << 20,
        ),
    )(xt, w, b_col)
    return out.reshape(B, C, H, W)


# speedup vs baseline: 12.4127x; 9.7647x over previous
import functools

import jax
import jax.numpy as jnp
from jax.experimental import pallas as pl
from jax.experimental.pallas import tpu as pltpu


def _se_body(x_ref, w_ref, b_ref, o_ref, *, inv_hw):
    x = x_ref[...]
    pooled = jnp.sum(x, axis=(1, 2)) * inv_hw
    logits = jax.lax.dot_general(
        pooled, w_ref[...], (((1,), (1,)), ((), ())),
        preferred_element_type=jnp.float32,
    ) + b_ref[...]
    gate = jax.nn.sigmoid(logits)
    o_ref[...] = x * gate[:, None, None, :]


def kernel(x, weight, bias):
    B, C, H, W = x.shape
    bblk = 4 if B % 4 == 0 else 1
    x_nhwc = jnp.transpose(x, (0, 2, 3, 1))
    w = jnp.asarray(weight).reshape(C, C)
    b_row = jnp.asarray(bias).reshape(1, C)

    out = pl.pallas_call(
        functools.partial(_se_body, inv_hw=1.0 / (H * W)),
        out_shape=jax.ShapeDtypeStruct((B, H, W, C), x.dtype),
        grid=(B // bblk,),
        in_specs=[
            pl.BlockSpec((bblk, H, W, C), lambda b: (b, 0, 0, 0)),
            pl.BlockSpec((C, C), lambda b: (0, 0)),
            pl.BlockSpec((1, C), lambda b: (0, 0)),
        ],
        out_specs=pl.BlockSpec((bblk, H, W, C), lambda b: (b, 0, 0, 0)),
        compiler_params=pltpu.CompilerParams(
            dimension_semantics=("parallel",),
            vmem_limit_bytes=56 << 20,
        ),
    )(x_nhwc, w, b_row)
    return jnp.transpose(out, (0, 3, 1, 2))
